# baseline (speedup 1.0000x reference)
import functools

import jax
import jax.numpy as jnp
from jax import lax
from jax.experimental import pallas as pl
from jax.experimental.pallas import tpu as pltpu
from jax.experimental.pallas import tpu_sc as plsc

N = 10000
D = 256
HID = 256
OUT = 64
MAXD = 17
G = 8

NC = 2
NS = 16
NW = NC * NS
PW = 320
NPAD = NW * PW
CHUNK = 16
NSUB = PW // CHUNK
GLANES = 32
NGRP = D // GLANES
IDXROW = 512

RBLK = 1024
NB = NPAD // RBLK


def _sc_gather_sum(table, idx4):
    mesh = plsc.VectorSubcoreMesh(core_axis_name="c", subcore_axis_name="s")
    DW = D // 2

    @functools.partial(
        pl.kernel,
        out_type=jax.ShapeDtypeStruct((NPAD, DW), jnp.int32),
        mesh=mesh,
        scratch_types=[
            pltpu.VMEM((NSUB, IDXROW), jnp.int32),
            pltpu.VMEM((2, MAXD * CHUNK, DW), jnp.int32),
            pltpu.VMEM((2, CHUNK, DW), jnp.int32),
            pltpu.SemaphoreType.DMA,
            pltpu.SemaphoreType.DMA,
            pltpu.SemaphoreType.DMA,
            pltpu.SemaphoreType.DMA,
        ],
    )
    def k(table_hbm, idx_hbm, out_hbm, idx_v, buf, obuf, g0, g1, o0, o1):
        wid = lax.axis_index("s") * NC + lax.axis_index("c")
        base = wid * PW
        gsem = (g0, g1)
        osem = (o0, o1)
        pltpu.sync_copy(idx_hbm.at[wid], idx_v)

        SPLITS = ((0, 128), (128, 128), (256, MAXD * CHUNK - 256))

        def fire(s, par, sem):
            for lo, n in SPLITS:
                pltpu.async_copy(table_hbm.at[idx_v.at[s, pl.ds(lo, n)]],
                                 buf.at[par, pl.ds(lo, n)], sem)

        def drain_gathers(par, sem):
            for lo, n in SPLITS:
                pltpu.make_async_copy(table_hbm.at[pl.ds(0, n)],
                                      buf.at[par, pl.ds(lo, n)], sem).wait()

        def drain_writeout(par, sem):
            pltpu.make_async_copy(table_hbm.at[pl.ds(0, CHUNK)],
                                  obuf.at[par], sem).wait()

        HMASK = jnp.uint32(0xFFFF0000)
        RND = jnp.uint32(0x8000)

        def tree(vs):
            while len(vs) > 1:
                nxt = [vs[m] + vs[m + 1] for m in range(0, len(vs) - 1, 2)]
                if len(vs) % 2:
                    nxt.append(vs[-1])
                vs = nxt
            return vs[0]

        def accumulate(par):
            def row(r, _):
                for c in range(NGRP):
                    sl = pl.ds(c * 16, 16)
                    us = [lax.bitcast_convert_type(buf[par, j * CHUNK + r, sl], jnp.uint32)
                          for j in range(MAXD)]
                    lo = tree([lax.bitcast_convert_type(u << 16, jnp.float32) for u in us])
                    hi = tree([lax.bitcast_convert_type(u & HMASK, jnp.float32) for u in us])
                    lo_u = (lax.bitcast_convert_type(lo, jnp.uint32) + RND) >> 16
                    hi_u = (lax.bitcast_convert_type(hi, jnp.uint32) + RND) & HMASK
                    obuf[par, r, sl] = lax.bitcast_convert_type(hi_u | lo_u, jnp.int32)
                return _
            lax.fori_loop(0, CHUNK, row, 0)

        def phase(i, s, par):
            nxt = s + 1

            @pl.when(nxt < NSUB)
            def _():
                fire(nxt, 1 - par, gsem[1 - par])

            with jax.named_scope("drain_g"):
                drain_gathers(par, gsem[par])

            @pl.when(i > 0)
            def _():
                drain_writeout(par, osem[par])

            with jax.named_scope("acc"):
                accumulate(par)
            pltpu.async_copy(obuf.at[par], out_hbm.at[pl.ds(base + s * CHUNK, CHUNK)],
                             osem[par])

        fire(0, 0, g0)

        def body(i, _):
            phase(i, 2 * i, 0)
            phase(i, 2 * i + 1, 1)
            return _

        lax.fori_loop(0, NSUB // 2, body, 0)
        drain_writeout(0, o0)
        drain_writeout(1, o1)

    return k(table, idx4)


def _tc_layer1(agg1, x_pad, segT, W0, b0):
    def body(agg_ref, x_ref, segT_ref, w_ref, b_ref, h_ref, ge0_ref, ge1_ref):
        i = pl.program_id(0)
        h = jnp.dot(agg_ref[...], w_ref[...], preferred_element_type=jnp.float32)
        h = jnp.maximum(h + b_ref[...], 0.0)
        h_ref[...] = h.astype(jnp.bfloat16)
        s = segT_ref[...]
        p0 = jnp.dot(s, x_ref[...], preferred_element_type=jnp.float32)
        p1 = jnp.dot(s, h, preferred_element_type=jnp.float32)

        @pl.when(i == 0)
        def _():
            ge0_ref[...] = p0
            ge1_ref[...] = p1

        @pl.when(i > 0)
        def _():
            ge0_ref[...] += p0
            ge1_ref[...] += p1

    return pl.pallas_call(
        body,
        grid=(NB,),
        in_specs=[
            pl.BlockSpec((RBLK, D), lambda i: (i, 0)),
            pl.BlockSpec((RBLK, D), lambda i: (i, 0)),
            pl.BlockSpec((G, RBLK), lambda i: (0, i)),
            pl.BlockSpec((D, HID), lambda i: (0, 0)),
            pl.BlockSpec((1, HID), lambda i: (0, 0)),
        ],
        out_specs=[
            pl.BlockSpec((RBLK, HID), lambda i: (i, 0)),
            pl.BlockSpec((G, D), lambda i: (0, 0)),
            pl.BlockSpec((G, HID), lambda i: (0, 0)),
        ],
        out_shape=[
            jax.ShapeDtypeStruct((NPAD, HID), jnp.bfloat16),
            jax.ShapeDtypeStruct((G, D), jnp.float32),
            jax.ShapeDtypeStruct((G, HID), jnp.float32),
        ],
    )(agg1, x_pad, segT, W0, b0)


def _tc_layer2(agg2, segT, W1, b1, ge0, ge1, C0w, C1w, C2w, cb):
    def body(agg_ref, segT_ref, w_ref, b_ref, ge0_ref, ge1_ref,
             c0_ref, c1_ref, c2_ref, cb_ref, preds_ref, acc_ref):
        i = pl.program_id(0)
        h = jnp.dot(agg_ref[...], w_ref[...], preferred_element_type=jnp.float32)
        h = jnp.maximum(h + b_ref[...], 0.0)
        p2 = jnp.dot(segT_ref[...], h, preferred_element_type=jnp.float32)

        @pl.when(i == 0)
        def _():
            acc_ref[...] = p2

        @pl.when(i > 0)
        def _():
            acc_ref[...] += p2

        @pl.when(i == NB - 1)
        def _():
            preds = jnp.dot(ge0_ref[...], c0_ref[...],
                            preferred_element_type=jnp.float32)
            preds += jnp.dot(ge1_ref[...], c1_ref[...],
                             preferred_element_type=jnp.float32)
            preds += jnp.dot(acc_ref[...], c2_ref[...],
                             preferred_element_type=jnp.float32)
            preds_ref[...] = preds + cb_ref[...]

    return pl.pallas_call(
        body,
        grid=(NB,),
        in_specs=[
            pl.BlockSpec((RBLK, HID), lambda i: (i, 0)),
            pl.BlockSpec((G, RBLK), lambda i: (0, i)),
            pl.BlockSpec((HID, HID), lambda i: (0, 0)),
            pl.BlockSpec((1, HID), lambda i: (0, 0)),
            pl.BlockSpec((G, D), lambda i: (0, 0)),
            pl.BlockSpec((G, HID), lambda i: (0, 0)),
            pl.BlockSpec((D, OUT), lambda i: (0, 0)),
            pl.BlockSpec((HID, OUT), lambda i: (0, 0)),
            pl.BlockSpec((HID, OUT), lambda i: (0, 0)),
            pl.BlockSpec((1, OUT), lambda i: (0, 0)),
        ],
        out_specs=pl.BlockSpec((G, OUT), lambda i: (0, 0)),
        out_shape=jax.ShapeDtypeStruct((G, OUT), jnp.float32),
        scratch_shapes=[pltpu.VMEM((G, HID), jnp.float32)],
    )(agg2, segT, W1, b1, ge0, ge1, C0w, C1w, C2w, cb)


def kernel(x, neighbours, segment_ids, W0, b0, W1, b1,
           C0w, C0b, C1w, C1b, C2w, C2b):
    x_pad = jnp.zeros((NPAD, D), jnp.float32).at[:N].set(x)
    spread = (jnp.arange((NPAD - N) * MAXD, dtype=jnp.int32) * 37) % N
    nbr_pad = jnp.concatenate(
        [neighbours, spread.reshape(NPAD - N, MAXD)], axis=0)
    idxp = nbr_pad.reshape(NW, NSUB, CHUNK, MAXD).transpose(0, 1, 3, 2)
    idx3 = jnp.zeros((NW, NSUB, IDXROW), jnp.int32).at[:, :, :MAXD * CHUNK].set(
        idxp.reshape(NW, NSUB, MAXD * CHUNK))
    seg_pad = jnp.full((NPAD,), G, jnp.int32).at[:N].set(segment_ids)
    segT = (seg_pad[None, :] == jnp.arange(G, dtype=jnp.int32)[:, None]
            ).astype(jnp.float32)

    def to_i32(a_bf):
        return lax.bitcast_convert_type(
            a_bf.reshape(NPAD, D // 2, 2), jnp.int32)

    def to_bf(a_i32):
        return lax.bitcast_convert_type(a_i32, jnp.bfloat16).reshape(NPAD, D)

    x_bf = x_pad.astype(jnp.bfloat16)
    agg1 = to_bf(_sc_gather_sum(to_i32(x_bf), idx3))
    H1, ge0, ge1 = _tc_layer1(agg1, x_pad, segT,
                              W0.astype(jnp.bfloat16), b0.reshape(1, HID))
    agg2 = to_bf(_sc_gather_sum(to_i32(H1), idx3))
    cb = (C0b + C1b + C2b).reshape(1, OUT)
    return _tc_layer2(agg2, segT, W1.astype(jnp.bfloat16), b1.reshape(1, HID),
                      ge0, ge1, C0w, C1w, C2w, cb)


# speedup vs baseline: 1.5940x; 1.5940x over previous
import functools

import jax
import jax.numpy as jnp
from jax import lax
from jax.experimental import pallas as pl
from jax.experimental.pallas import tpu as pltpu
from jax.experimental.pallas import tpu_sc as plsc

N = 10000
D = 256
HID = 256
OUT = 64
MAXD = 17
G = 8

NC = 2
NS = 16
NW = NC * NS
PW = 320
NPAD = NW * PW
CHUNK = 16
NSUB = PW // CHUNK
GLANES = 32
NGRP = D // GLANES
IDXROW = 512

RBLK = 1024
NB = NPAD // RBLK


def _sc_gather_sum(table, idx4):
    mesh = plsc.VectorSubcoreMesh(core_axis_name="c", subcore_axis_name="s")
    DW = D // 2

    @functools.partial(
        pl.kernel,
        out_type=jax.ShapeDtypeStruct((NPAD, DW), jnp.int32),
        mesh=mesh,
        scratch_types=[
            pltpu.VMEM((NSUB, IDXROW), jnp.int32),
            pltpu.VMEM((2, MAXD * CHUNK, DW), jnp.int32),
            pltpu.VMEM((2, CHUNK, DW), jnp.int32),
            pltpu.SemaphoreType.DMA,
            pltpu.SemaphoreType.DMA,
            pltpu.SemaphoreType.DMA,
            pltpu.SemaphoreType.DMA,
        ],
    )
    def k(table_hbm, idx_hbm, out_hbm, idx_v, buf, obuf, g0, g1, o0, o1):
        wid = lax.axis_index("s") * NC + lax.axis_index("c")
        base = wid * PW
        gsem = (g0, g1)
        osem = (o0, o1)
        pltpu.sync_copy(idx_hbm.at[wid], idx_v)

        SPLITS = ((0, 128), (128, 128), (256, MAXD * CHUNK - 256))

        def fire(s, par, sem):
            for lo, n in SPLITS:
                pltpu.async_copy(table_hbm.at[idx_v.at[s, pl.ds(lo, n)]],
                                 buf.at[par, pl.ds(lo, n)], sem)

        def drain_gathers(par, sem):
            for lo, n in SPLITS:
                pltpu.make_async_copy(table_hbm.at[pl.ds(0, n)],
                                      buf.at[par, pl.ds(lo, n)], sem).wait()

        def drain_writeout(par, sem):
            pltpu.make_async_copy(table_hbm.at[pl.ds(0, CHUNK)],
                                  obuf.at[par], sem).wait()

        HMASK = jnp.uint32(0xFFFF0000)
        RND = jnp.uint32(0x8000)

        def tree(vs):
            while len(vs) > 1:
                nxt = [vs[m] + vs[m + 1] for m in range(0, len(vs) - 1, 2)]
                if len(vs) % 2:
                    nxt.append(vs[-1])
                vs = nxt
            return vs[0]

        def accumulate(par):
            def row(r, _):
                for c in range(NGRP):
                    sl = pl.ds(c * 16, 16)

                    def halves(j):
                        u = lax.bitcast_convert_type(
                            buf[par, j * CHUNK + r, sl], jnp.uint32)
                        return (lax.bitcast_convert_type(u << 16, jnp.float32),
                                lax.bitcast_convert_type(u & HMASK, jnp.float32))

                    lo, hi = halves(0)
                    for j in range(1, MAXD):
                        l2, h2 = halves(j)
                        lo = lo + l2
                        hi = hi + h2
                    lo_u = (lax.bitcast_convert_type(lo, jnp.uint32) + RND) >> 16
                    hi_u = (lax.bitcast_convert_type(hi, jnp.uint32) + RND) & HMASK
                    obuf[par, r, sl] = lax.bitcast_convert_type(hi_u | lo_u, jnp.int32)
                return _
            lax.fori_loop(0, CHUNK, row, 0)

        def phase(i, s, par):
            nxt = s + 1

            @pl.when(nxt < NSUB)
            def _():
                fire(nxt, 1 - par, gsem[1 - par])

            with jax.named_scope("drain_g"):
                drain_gathers(par, gsem[par])

            @pl.when(i > 0)
            def _():
                drain_writeout(par, osem[par])

            with jax.named_scope("acc"):
                accumulate(par)
            pltpu.async_copy(obuf.at[par], out_hbm.at[pl.ds(base + s * CHUNK, CHUNK)],
                             osem[par])

        fire(0, 0, g0)

        def body(i, _):
            phase(i, 2 * i, 0)
            phase(i, 2 * i + 1, 1)
            return _

        lax.fori_loop(0, NSUB // 2, body, 0)
        drain_writeout(0, o0)
        drain_writeout(1, o1)

    return k(table, idx4)


def _tc_layer1(agg1, x_pad, segT, W0, b0):
    def body(agg_ref, x_ref, segT_ref, w_ref, b_ref, h_ref, ge0_ref, ge1_ref):
        i = pl.program_id(0)
        h = jnp.dot(agg_ref[...], w_ref[...], preferred_element_type=jnp.float32)
        h = jnp.maximum(h + b_ref[...], 0.0)
        h_ref[...] = h.astype(jnp.bfloat16)
        s = segT_ref[...]
        p0 = jnp.dot(s, x_ref[...], preferred_element_type=jnp.float32)
        p1 = jnp.dot(s, h, preferred_element_type=jnp.float32)

        @pl.when(i == 0)
        def _():
            ge0_ref[...] = p0
            ge1_ref[...] = p1

        @pl.when(i > 0)
        def _():
            ge0_ref[...] += p0
            ge1_ref[...] += p1

    return pl.pallas_call(
        body,
        grid=(NB,),
        in_specs=[
            pl.BlockSpec((RBLK, D), lambda i: (i, 0)),
            pl.BlockSpec((RBLK, D), lambda i: (i, 0)),
            pl.BlockSpec((G, RBLK), lambda i: (0, i)),
            pl.BlockSpec((D, HID), lambda i: (0, 0)),
            pl.BlockSpec((1, HID), lambda i: (0, 0)),
        ],
        out_specs=[
            pl.BlockSpec((RBLK, HID), lambda i: (i, 0)),
            pl.BlockSpec((G, D), lambda i: (0, 0)),
            pl.BlockSpec((G, HID), lambda i: (0, 0)),
        ],
        out_shape=[
            jax.ShapeDtypeStruct((NPAD, HID), jnp.bfloat16),
            jax.ShapeDtypeStruct((G, D), jnp.float32),
            jax.ShapeDtypeStruct((G, HID), jnp.float32),
        ],
    )(agg1, x_pad, segT, W0, b0)


def _tc_layer2(agg2, segT, W1, b1, ge0, ge1, C0w, C1w, C2w, cb):
    def body(agg_ref, segT_ref, w_ref, b_ref, ge0_ref, ge1_ref,
             c0_ref, c1_ref, c2_ref, cb_ref, preds_ref, acc_ref):
        i = pl.program_id(0)
        h = jnp.dot(agg_ref[...], w_ref[...], preferred_element_type=jnp.float32)
        h = jnp.maximum(h + b_ref[...], 0.0)
        p2 = jnp.dot(segT_ref[...], h, preferred_element_type=jnp.float32)

        @pl.when(i == 0)
        def _():
            acc_ref[...] = p2

        @pl.when(i > 0)
        def _():
            acc_ref[...] += p2

        @pl.when(i == NB - 1)
        def _():
            preds = jnp.dot(ge0_ref[...], c0_ref[...],
                            preferred_element_type=jnp.float32)
            preds += jnp.dot(ge1_ref[...], c1_ref[...],
                             preferred_element_type=jnp.float32)
            preds += jnp.dot(acc_ref[...], c2_ref[...],
                             preferred_element_type=jnp.float32)
            preds_ref[...] = preds + cb_ref[...]

    return pl.pallas_call(
        body,
        grid=(NB,),
        in_specs=[
            pl.BlockSpec((RBLK, HID), lambda i: (i, 0)),
            pl.BlockSpec((G, RBLK), lambda i: (0, i)),
            pl.BlockSpec((HID, HID), lambda i: (0, 0)),
            pl.BlockSpec((1, HID), lambda i: (0, 0)),
            pl.BlockSpec((G, D), lambda i: (0, 0)),
            pl.BlockSpec((G, HID), lambda i: (0, 0)),
            pl.BlockSpec((D, OUT), lambda i: (0, 0)),
            pl.BlockSpec((HID, OUT), lambda i: (0, 0)),
            pl.BlockSpec((HID, OUT), lambda i: (0, 0)),
            pl.BlockSpec((1, OUT), lambda i: (0, 0)),
        ],
        out_specs=pl.BlockSpec((G, OUT), lambda i: (0, 0)),
        out_shape=jax.ShapeDtypeStruct((G, OUT), jnp.float32),
        scratch_shapes=[pltpu.VMEM((G, HID), jnp.float32)],
    )(agg2, segT, W1, b1, ge0, ge1, C0w, C1w, C2w, cb)


def kernel(x, neighbours, segment_ids, W0, b0, W1, b1,
           C0w, C0b, C1w, C1b, C2w, C2b):
    x_pad = jnp.zeros((NPAD, D), jnp.float32).at[:N].set(x)
    spread = (jnp.arange((NPAD - N) * MAXD, dtype=jnp.int32) * 37) % N
    nbr_pad = jnp.concatenate(
        [neighbours, spread.reshape(NPAD - N, MAXD)], axis=0)
    idxp = nbr_pad.reshape(NW, NSUB, CHUNK, MAXD).transpose(0, 1, 3, 2)
    idx3 = jnp.zeros((NW, NSUB, IDXROW), jnp.int32).at[:, :, :MAXD * CHUNK].set(
        idxp.reshape(NW, NSUB, MAXD * CHUNK))
    seg_pad = jnp.full((NPAD,), G, jnp.int32).at[:N].set(segment_ids)
    segT = (seg_pad[None, :] == jnp.arange(G, dtype=jnp.int32)[:, None]
            ).astype(jnp.float32)

    def to_i32(a_bf):
        return lax.bitcast_convert_type(
            a_bf.reshape(NPAD, D // 2, 2), jnp.int32)

    def to_bf(a_i32):
        return lax.bitcast_convert_type(a_i32, jnp.bfloat16).reshape(NPAD, D)

    x_bf = x_pad.astype(jnp.bfloat16)
    agg1 = to_bf(_sc_gather_sum(to_i32(x_bf), idx3))
    H1, ge0, ge1 = _tc_layer1(agg1, x_pad, segT,
                              W0.astype(jnp.bfloat16), b0.reshape(1, HID))
    agg2 = to_bf(_sc_gather_sum(to_i32(H1), idx3))
    cb = (C0b + C1b + C2b).reshape(1, OUT)
    return _tc_layer2(agg2, segT, W1.astype(jnp.bfloat16), b1.reshape(1, HID),
                      ge0, ge1, C0w, C1w, C2w, cb)


# speedup vs baseline: 3.8994x; 2.4463x over previous
import functools

import jax
import jax.numpy as jnp
import numpy as np
from jax import lax
from jax.experimental import pallas as pl
from jax.experimental.pallas import tpu as pltpu
from jax.experimental.pallas import tpu_sc as plsc

N = 10000
D = 256
HID = 256
OUT = 64
MAXD = 17
G = 8

NC = 2
NS = 16
NW = NC * NS
PW = 320
NPAD = NW * PW
CHUNK = 16
NSUB = PW // CHUNK
GLANES = 32
NGRP = D // GLANES
IDXROW = 512

RBLK = 1024
NB = NPAD // RBLK


def _sc_gather_sum(table, idx4):
    mesh = plsc.VectorSubcoreMesh(core_axis_name="c", subcore_axis_name="s")
    DW = D // 2

    @functools.partial(
        pl.kernel,
        out_type=jax.ShapeDtypeStruct((NPAD, DW), jnp.int32),
        mesh=mesh,
        scratch_types=[
            pltpu.VMEM((NSUB, IDXROW), jnp.int32),
            pltpu.VMEM((2, MAXD * CHUNK, DW), jnp.int32),
            pltpu.VMEM((2, CHUNK, DW), jnp.int32),
            pltpu.SemaphoreType.DMA,
            pltpu.SemaphoreType.DMA,
            pltpu.SemaphoreType.DMA,
            pltpu.SemaphoreType.DMA,
        ],
    )
    def k(table_hbm, idx_hbm, out_hbm, idx_v, buf, obuf, g0, g1, o0, o1):
        wid = lax.axis_index("s") * NC + lax.axis_index("c")
        base = wid * PW
        gsem = (g0, g1)
        osem = (o0, o1)
        pltpu.sync_copy(idx_hbm.at[wid], idx_v)

        SPLITS = ((0, 128), (128, 128), (256, MAXD * CHUNK - 256))

        def fire(s, par, sem):
            for lo, n in SPLITS:
                pltpu.async_copy(table_hbm.at[idx_v.at[s, pl.ds(lo, n)]],
                                 buf.at[par, pl.ds(lo, n)], sem)

        def drain_gathers(par, sem):
            for lo, n in SPLITS:
                pltpu.make_async_copy(table_hbm.at[pl.ds(0, n)],
                                      buf.at[par, pl.ds(lo, n)], sem).wait()

        def drain_writeout(par, sem):
            pltpu.make_async_copy(table_hbm.at[pl.ds(0, CHUNK)],
                                  obuf.at[par], sem).wait()

        HMASK = jnp.uint32(0xFFFF0000)
        RND = jnp.uint32(0x8000)

        def tree(vs):
            while len(vs) > 1:
                nxt = [vs[m] + vs[m + 1] for m in range(0, len(vs) - 1, 2)]
                if len(vs) % 2:
                    nxt.append(vs[-1])
                vs = nxt
            return vs[0]

        def accumulate(par):
            def row(r, _):
                for c in range(NGRP):
                    sl = pl.ds(c * 16, 16)

                    def halves(j):
                        u = lax.bitcast_convert_type(
                            buf[par, j * CHUNK + r, sl], jnp.uint32)
                        return (lax.bitcast_convert_type(u << 16, jnp.float32),
                                lax.bitcast_convert_type(u & HMASK, jnp.float32))

                    lo, hi = halves(0)
                    for j in range(1, MAXD):
                        l2, h2 = halves(j)
                        lo = lo + l2
                        hi = hi + h2
                    lo_u = (lax.bitcast_convert_type(lo, jnp.uint32) + RND) >> 16
                    hi_u = (lax.bitcast_convert_type(hi, jnp.uint32) + RND) & HMASK
                    obuf[par, r, sl] = lax.bitcast_convert_type(hi_u | lo_u, jnp.int32)
                return _
            lax.fori_loop(0, CHUNK, row, 0)

        def phase(i, s, par):
            nxt = s + 1

            @pl.when(nxt < NSUB)
            def _():
                fire(nxt, 1 - par, gsem[1 - par])

            with jax.named_scope("drain_g"):
                drain_gathers(par, gsem[par])

            @pl.when(i > 0)
            def _():
                drain_writeout(par, osem[par])

            with jax.named_scope("acc"):
                accumulate(par)
            pltpu.async_copy(obuf.at[par], out_hbm.at[pl.ds(base + s * CHUNK, CHUNK)],
                             osem[par])

        fire(0, 0, g0)

        def body(i, _):
            phase(i, 2 * i, 0)
            phase(i, 2 * i + 1, 1)
            return _

        lax.fori_loop(0, NSUB // 2, body, 0)
        drain_writeout(0, o0)
        drain_writeout(1, o1)

    return k(table, idx4)


def _pack_words(h):
    u = lax.bitcast_convert_type(h, jnp.uint32)
    lo = (u[:, :D // 2] + np.uint32(0x8000)) >> 16
    hi = (u[:, D // 2:] + np.uint32(0x8000)) & np.uint32(0xFFFF0000)
    return lax.bitcast_convert_type(hi | lo, jnp.int32)


def _unpack_words(w):
    u = lax.bitcast_convert_type(w, jnp.uint32)
    lo = lax.bitcast_convert_type(u << 16, jnp.float32)
    hi = lax.bitcast_convert_type(u & np.uint32(0xFFFF0000), jnp.float32)
    return jnp.concatenate([lo, hi], axis=1)


def _tc_pack_x(x_pad):
    def body(x_ref, o_ref):
        o_ref[...] = _pack_words(x_ref[...])

    return pl.pallas_call(
        body,
        grid=(NB,),
        in_specs=[pl.BlockSpec((RBLK, D), lambda i: (i, 0))],
        out_specs=pl.BlockSpec((RBLK, D // 2), lambda i: (i, 0)),
        out_shape=jax.ShapeDtypeStruct((NPAD, D // 2), jnp.int32),
    )(x_pad)


def _tc_layer1(agg1, x_pad, segT, W0, b0):
    def body(agg_ref, x_ref, segT_ref, w_ref, b_ref, h_ref, ge0_ref, ge1_ref):
        i = pl.program_id(0)
        agg = _unpack_words(agg_ref[...])
        h = jnp.dot(agg, w_ref[...], preferred_element_type=jnp.float32)
        h = jnp.maximum(h + b_ref[...], 0.0)
        h_ref[...] = _pack_words(h)
        s = segT_ref[...]
        p0 = jnp.dot(s, x_ref[...], preferred_element_type=jnp.float32)
        p1 = jnp.dot(s, h, preferred_element_type=jnp.float32)

        @pl.when(i == 0)
        def _():
            ge0_ref[...] = p0
            ge1_ref[...] = p1

        @pl.when(i > 0)
        def _():
            ge0_ref[...] += p0
            ge1_ref[...] += p1

    return pl.pallas_call(
        body,
        grid=(NB,),
        in_specs=[
            pl.BlockSpec((RBLK, D // 2), lambda i: (i, 0)),
            pl.BlockSpec((RBLK, D), lambda i: (i, 0)),
            pl.BlockSpec((G, RBLK), lambda i: (0, i)),
            pl.BlockSpec((D, HID), lambda i: (0, 0)),
            pl.BlockSpec((1, HID), lambda i: (0, 0)),
        ],
        out_specs=[
            pl.BlockSpec((RBLK, HID // 2), lambda i: (i, 0)),
            pl.BlockSpec((G, D), lambda i: (0, 0)),
            pl.BlockSpec((G, HID), lambda i: (0, 0)),
        ],
        out_shape=[
            jax.ShapeDtypeStruct((NPAD, HID // 2), jnp.int32),
            jax.ShapeDtypeStruct((G, D), jnp.float32),
            jax.ShapeDtypeStruct((G, HID), jnp.float32),
        ],
    )(agg1, x_pad, segT, W0, b0)


def _tc_layer2(agg2, segT, W1, b1, ge0, ge1, C0w, C1w, C2w, cb):
    def body(agg_ref, segT_ref, w_ref, b_ref, ge0_ref, ge1_ref,
             c0_ref, c1_ref, c2_ref, cb_ref, preds_ref, acc_ref):
        i = pl.program_id(0)
        agg = _unpack_words(agg_ref[...])
        h = jnp.dot(agg, w_ref[...], preferred_element_type=jnp.float32)
        h = jnp.maximum(h + b_ref[...], 0.0)
        p2 = jnp.dot(segT_ref[...], h, preferred_element_type=jnp.float32)

        @pl.when(i == 0)
        def _():
            acc_ref[...] = p2

        @pl.when(i > 0)
        def _():
            acc_ref[...] += p2

        @pl.when(i == NB - 1)
        def _():
            preds = jnp.dot(ge0_ref[...], c0_ref[...],
                            preferred_element_type=jnp.float32)
            preds += jnp.dot(ge1_ref[...], c1_ref[...],
                             preferred_element_type=jnp.float32)
            preds += jnp.dot(acc_ref[...], c2_ref[...],
                             preferred_element_type=jnp.float32)
            preds_ref[...] = preds + cb_ref[...]

    return pl.pallas_call(
        body,
        grid=(NB,),
        in_specs=[
            pl.BlockSpec((RBLK, HID // 2), lambda i: (i, 0)),
            pl.BlockSpec((G, RBLK), lambda i: (0, i)),
            pl.BlockSpec((HID, HID), lambda i: (0, 0)),
            pl.BlockSpec((1, HID), lambda i: (0, 0)),
            pl.BlockSpec((G, D), lambda i: (0, 0)),
            pl.BlockSpec((G, HID), lambda i: (0, 0)),
            pl.BlockSpec((D, OUT), lambda i: (0, 0)),
            pl.BlockSpec((HID, OUT), lambda i: (0, 0)),
            pl.BlockSpec((HID, OUT), lambda i: (0, 0)),
            pl.BlockSpec((1, OUT), lambda i: (0, 0)),
        ],
        out_specs=pl.BlockSpec((G, OUT), lambda i: (0, 0)),
        out_shape=jax.ShapeDtypeStruct((G, OUT), jnp.float32),
        scratch_shapes=[pltpu.VMEM((G, HID), jnp.float32)],
    )(agg2, segT, W1, b1, ge0, ge1, C0w, C1w, C2w, cb)


def kernel(x, neighbours, segment_ids, W0, b0, W1, b1,
           C0w, C0b, C1w, C1b, C2w, C2b):
    x_pad = jnp.zeros((NPAD, D), jnp.float32).at[:N].set(x)
    spread = (jnp.arange((NPAD - N) * MAXD, dtype=jnp.int32) * 37) % N
    nbr_pad = jnp.concatenate(
        [neighbours, spread.reshape(NPAD - N, MAXD)], axis=0)
    idxp = nbr_pad.reshape(NW, NSUB, CHUNK, MAXD).transpose(0, 1, 3, 2)
    idx3 = jnp.zeros((NW, NSUB, IDXROW), jnp.int32).at[:, :, :MAXD * CHUNK].set(
        idxp.reshape(NW, NSUB, MAXD * CHUNK))
    seg_pad = jnp.full((NPAD,), G, jnp.int32).at[:N].set(segment_ids)
    segT = (seg_pad[None, :] == jnp.arange(G, dtype=jnp.int32)[:, None]
            ).astype(jnp.float32)

    xw = _tc_pack_x(x_pad)
    agg1 = _sc_gather_sum(xw, idx3)
    H1w, ge0, ge1 = _tc_layer1(agg1, x_pad, segT, W0, b0.reshape(1, HID))
    agg2 = _sc_gather_sum(H1w, idx3)
    cb = (C0b + C1b + C2b).reshape(1, OUT)
    return _tc_layer2(agg2, segT, W1, b1.reshape(1, HID),
                      ge0, ge1, C0w, C1w, C2w, cb)


# speedup vs baseline: 4.0903x; 1.0490x over previous
import functools

import jax
import jax.numpy as jnp
import numpy as np
from jax import lax
from jax.experimental import pallas as pl
from jax.experimental.pallas import tpu as pltpu
from jax.experimental.pallas import tpu_sc as plsc

N = 10000
D = 256
HID = 256
OUT = 64
MAXD = 17
G = 8

NC = 2
NS = 16
NW = NC * NS
PW = 320
NPAD = NW * PW
CHUNK = 16
NSUB = PW // CHUNK
GLANES = 32
NGRP = D // GLANES
IDXROW = 512

RBLK = 1024
NB = NPAD // RBLK


def _sc_gather_sum(table, idx4):
    mesh = plsc.VectorSubcoreMesh(core_axis_name="c", subcore_axis_name="s")
    DW = D // 2

    @functools.partial(
        pl.kernel,
        out_type=jax.ShapeDtypeStruct((NPAD, DW), jnp.int32),
        mesh=mesh,
        scratch_types=[
            pltpu.VMEM((NSUB, IDXROW), jnp.int32),
            pltpu.VMEM((2, MAXD * CHUNK, DW), jnp.int32),
            pltpu.VMEM((2, CHUNK, DW), jnp.int32),
            pltpu.SemaphoreType.DMA,
            pltpu.SemaphoreType.DMA,
            pltpu.SemaphoreType.DMA,
            pltpu.SemaphoreType.DMA,
        ],
    )
    def k(table_hbm, idx_hbm, out_hbm, idx_v, buf, obuf, g0, g1, o0, o1):
        wid = lax.axis_index("s") * NC + lax.axis_index("c")
        base = wid * PW
        gsem = (g0, g1)
        osem = (o0, o1)
        pltpu.sync_copy(idx_hbm.at[wid], idx_v)

        SPLITS = ((0, 128), (128, 128), (256, MAXD * CHUNK - 256))

        def fire(s, par, sem):
            for lo, n in SPLITS:
                pltpu.async_copy(table_hbm.at[idx_v.at[s, pl.ds(lo, n)]],
                                 buf.at[par, pl.ds(lo, n)], sem)

        def drain_gathers(par, sem):
            for lo, n in SPLITS:
                pltpu.make_async_copy(table_hbm.at[pl.ds(0, n)],
                                      buf.at[par, pl.ds(lo, n)], sem).wait()

        def drain_writeout(par, sem):
            pltpu.make_async_copy(table_hbm.at[pl.ds(0, CHUNK)],
                                  obuf.at[par], sem).wait()

        HMASK = jnp.uint32(0xFFFF0000)
        RND = jnp.uint32(0x8000)

        def tree(vs):
            while len(vs) > 1:
                nxt = [vs[m] + vs[m + 1] for m in range(0, len(vs) - 1, 2)]
                if len(vs) % 2:
                    nxt.append(vs[-1])
                vs = nxt
            return vs[0]

        def accumulate(par):
            def row(r, _):
                for c in range(NGRP):
                    sl = pl.ds(c * 16, 16)

                    def halves(j):
                        u = lax.bitcast_convert_type(
                            buf[par, j * CHUNK + r, sl], jnp.uint32)
                        return (lax.bitcast_convert_type(u << 16, jnp.float32),
                                lax.bitcast_convert_type(u, jnp.float32))

                    lo, hi = halves(0)
                    for j in range(1, MAXD):
                        l2, h2 = halves(j)
                        lo = lo + l2
                        hi = hi + h2
                    lo_u = (lax.bitcast_convert_type(lo, jnp.uint32) + RND) >> 16
                    hi_u = (lax.bitcast_convert_type(hi, jnp.uint32) + RND) & HMASK
                    obuf[par, r, sl] = lax.bitcast_convert_type(hi_u | lo_u, jnp.int32)
                return _
            lax.fori_loop(0, CHUNK, row, 0)

        def phase(i, s, par):
            nxt = s + 1

            @pl.when(nxt < NSUB)
            def _():
                fire(nxt, 1 - par, gsem[1 - par])

            with jax.named_scope("drain_g"):
                drain_gathers(par, gsem[par])

            @pl.when(i > 0)
            def _():
                drain_writeout(par, osem[par])

            with jax.named_scope("acc"):
                accumulate(par)
            pltpu.async_copy(obuf.at[par], out_hbm.at[pl.ds(base + s * CHUNK, CHUNK)],
                             osem[par])

        fire(0, 0, g0)

        def body(i, _):
            phase(i, 2 * i, 0)
            phase(i, 2 * i + 1, 1)
            return _

        lax.fori_loop(0, NSUB // 2, body, 0)
        drain_writeout(0, o0)
        drain_writeout(1, o1)

    return k(table, idx4)


def _pack_words(h):
    u = lax.bitcast_convert_type(h, jnp.uint32)
    lo = (u[:, :D // 2] + np.uint32(0x8000)) >> 16
    hi = (u[:, D // 2:] + np.uint32(0x8000)) & np.uint32(0xFFFF0000)
    return lax.bitcast_convert_type(hi | lo, jnp.int32)


def _unpack_words(w):
    u = lax.bitcast_convert_type(w, jnp.uint32)
    lo = lax.bitcast_convert_type(u << 16, jnp.float32)
    hi = lax.bitcast_convert_type(u & np.uint32(0xFFFF0000), jnp.float32)
    return jnp.concatenate([lo, hi], axis=1)


def _tc_pack_x(x_pad):
    def body(x_ref, o_ref):
        o_ref[...] = _pack_words(x_ref[...])

    return pl.pallas_call(
        body,
        grid=(NB,),
        in_specs=[pl.BlockSpec((RBLK, D), lambda i: (i, 0))],
        out_specs=pl.BlockSpec((RBLK, D // 2), lambda i: (i, 0)),
        out_shape=jax.ShapeDtypeStruct((NPAD, D // 2), jnp.int32),
    )(x_pad)


def _tc_layer1(agg1, x_pad, segT, W0, b0):
    def body(agg_ref, x_ref, segT_ref, w_ref, b_ref, h_ref, ge0_ref, ge1_ref):
        i = pl.program_id(0)
        agg = _unpack_words(agg_ref[...])
        h = jnp.dot(agg, w_ref[...], preferred_element_type=jnp.float32)
        h = jnp.maximum(h + b_ref[...], 0.0)
        h_ref[...] = _pack_words(h)
        s = segT_ref[...]
        p0 = jnp.dot(s, x_ref[...], preferred_element_type=jnp.float32)
        p1 = jnp.dot(s, h, preferred_element_type=jnp.float32)

        @pl.when(i == 0)
        def _():
            ge0_ref[...] = p0
            ge1_ref[...] = p1

        @pl.when(i > 0)
        def _():
            ge0_ref[...] += p0
            ge1_ref[...] += p1

    return pl.pallas_call(
        body,
        grid=(NB,),
        in_specs=[
            pl.BlockSpec((RBLK, D // 2), lambda i: (i, 0)),
            pl.BlockSpec((RBLK, D), lambda i: (i, 0)),
            pl.BlockSpec((G, RBLK), lambda i: (0, i)),
            pl.BlockSpec((D, HID), lambda i: (0, 0)),
            pl.BlockSpec((1, HID), lambda i: (0, 0)),
        ],
        out_specs=[
            pl.BlockSpec((RBLK, HID // 2), lambda i: (i, 0)),
            pl.BlockSpec((G, D), lambda i: (0, 0)),
            pl.BlockSpec((G, HID), lambda i: (0, 0)),
        ],
        out_shape=[
            jax.ShapeDtypeStruct((NPAD, HID // 2), jnp.int32),
            jax.ShapeDtypeStruct((G, D), jnp.float32),
            jax.ShapeDtypeStruct((G, HID), jnp.float32),
        ],
    )(agg1, x_pad, segT, W0, b0)


def _tc_layer2(agg2, segT, W1, b1, ge0, ge1, C0w, C1w, C2w, cb):
    def body(agg_ref, segT_ref, w_ref, b_ref, ge0_ref, ge1_ref,
             c0_ref, c1_ref, c2_ref, cb_ref, preds_ref, acc_ref):
        i = pl.program_id(0)
        agg = _unpack_words(agg_ref[...])
        h = jnp.dot(agg, w_ref[...], preferred_element_type=jnp.float32)
        h = jnp.maximum(h + b_ref[...], 0.0)
        p2 = jnp.dot(segT_ref[...], h, preferred_element_type=jnp.float32)

        @pl.when(i == 0)
        def _():
            acc_ref[...] = p2

        @pl.when(i > 0)
        def _():
            acc_ref[...] += p2

        @pl.when(i == NB - 1)
        def _():
            preds = jnp.dot(ge0_ref[...], c0_ref[...],
                            preferred_element_type=jnp.float32)
            preds += jnp.dot(ge1_ref[...], c1_ref[...],
                             preferred_element_type=jnp.float32)
            preds += jnp.dot(acc_ref[...], c2_ref[...],
                             preferred_element_type=jnp.float32)
            preds_ref[...] = preds + cb_ref[...]

    return pl.pallas_call(
        body,
        grid=(NB,),
        in_specs=[
            pl.BlockSpec((RBLK, HID // 2), lambda i: (i, 0)),
            pl.BlockSpec((G, RBLK), lambda i: (0, i)),
            pl.BlockSpec((HID, HID), lambda i: (0, 0)),
            pl.BlockSpec((1, HID), lambda i: (0, 0)),
            pl.BlockSpec((G, D), lambda i: (0, 0)),
            pl.BlockSpec((G, HID), lambda i: (0, 0)),
            pl.BlockSpec((D, OUT), lambda i: (0, 0)),
            pl.BlockSpec((HID, OUT), lambda i: (0, 0)),
            pl.BlockSpec((HID, OUT), lambda i: (0, 0)),
            pl.BlockSpec((1, OUT), lambda i: (0, 0)),
        ],
        out_specs=pl.BlockSpec((G, OUT), lambda i: (0, 0)),
        out_shape=jax.ShapeDtypeStruct((G, OUT), jnp.float32),
        scratch_shapes=[pltpu.VMEM((G, HID), jnp.float32)],
    )(agg2, segT, W1, b1, ge0, ge1, C0w, C1w, C2w, cb)


def kernel(x, neighbours, segment_ids, W0, b0, W1, b1,
           C0w, C0b, C1w, C1b, C2w, C2b):
    x_pad = jnp.zeros((NPAD, D), jnp.float32).at[:N].set(x)
    spread = (jnp.arange((NPAD - N) * MAXD, dtype=jnp.int32) * 37) % N
    nbr_pad = jnp.concatenate(
        [neighbours, spread.reshape(NPAD - N, MAXD)], axis=0)
    idxp = nbr_pad.reshape(NW, NSUB, CHUNK, MAXD).transpose(0, 1, 3, 2)
    idx3 = jnp.zeros((NW, NSUB, IDXROW), jnp.int32).at[:, :, :MAXD * CHUNK].set(
        idxp.reshape(NW, NSUB, MAXD * CHUNK))
    seg_pad = jnp.full((NPAD,), G, jnp.int32).at[:N].set(segment_ids)
    segT = (seg_pad[None, :] == jnp.arange(G, dtype=jnp.int32)[:, None]
            ).astype(jnp.float32)

    xw = _tc_pack_x(x_pad)
    agg1 = _sc_gather_sum(xw, idx3)
    H1w, ge0, ge1 = _tc_layer1(agg1, x_pad, segT, W0, b0.reshape(1, HID))
    agg2 = _sc_gather_sum(H1w, idx3)
    cb = (C0b + C1b + C2b).reshape(1, OUT)
    return _tc_layer2(agg2, segT, W1, b1.reshape(1, HID),
                      ge0, ge1, C0w, C1w, C2w, cb)


# speedup vs baseline: 4.0919x; 1.0004x over previous
import functools

import jax
import jax.numpy as jnp
import numpy as np
from jax import lax
from jax.experimental import pallas as pl
from jax.experimental.pallas import tpu as pltpu
from jax.experimental.pallas import tpu_sc as plsc

N = 10000
D = 256
HID = 256
OUT = 64
MAXD = 17
G = 8

NC = 2
NS = 16
NW = NC * NS
PW = 320
NPAD = NW * PW
CHUNK = 16
NSUB = PW // CHUNK
GLANES = 32
NGRP = D // GLANES
IDXROW = 512

RBLK = 1024
NB = NPAD // RBLK


def _sc_gather_sum(table, idx4):
    mesh = plsc.VectorSubcoreMesh(core_axis_name="c", subcore_axis_name="s")
    DW = D // 2

    @functools.partial(
        pl.kernel,
        out_type=jax.ShapeDtypeStruct((NPAD, DW), jnp.int32),
        mesh=mesh,
        scratch_types=[
            pltpu.VMEM((NSUB, IDXROW), jnp.int32),
            pltpu.VMEM((2, MAXD * CHUNK, DW), jnp.int32),
            pltpu.VMEM((2, CHUNK, DW), jnp.int32),
            pltpu.SemaphoreType.DMA,
            pltpu.SemaphoreType.DMA,
            pltpu.SemaphoreType.DMA,
            pltpu.SemaphoreType.DMA,
        ],
    )
    def k(table_hbm, idx_hbm, out_hbm, idx_v, buf, obuf, g0, g1, o0, o1):
        wid = lax.axis_index("s") * NC + lax.axis_index("c")
        base = wid * PW
        gsem = (g0, g1)
        osem = (o0, o1)
        pltpu.sync_copy(idx_hbm.at[wid], idx_v)

        SPLITS = ((0, 128), (128, 128), (256, MAXD * CHUNK - 256))

        def fire(s, par, sem):
            for lo, n in SPLITS:
                pltpu.async_copy(table_hbm.at[idx_v.at[s, pl.ds(lo, n)]],
                                 buf.at[par, pl.ds(lo, n)], sem)

        def drain_gathers(par, sem):
            for lo, n in SPLITS:
                pltpu.make_async_copy(table_hbm.at[pl.ds(0, n)],
                                      buf.at[par, pl.ds(lo, n)], sem).wait()

        def drain_writeout(par, sem):
            pltpu.make_async_copy(table_hbm.at[pl.ds(0, CHUNK)],
                                  obuf.at[par], sem).wait()

        HMASK = jnp.uint32(0xFFFF0000)
        RND = jnp.uint32(0x8000)

        def tree(vs):
            while len(vs) > 1:
                nxt = [vs[m] + vs[m + 1] for m in range(0, len(vs) - 1, 2)]
                if len(vs) % 2:
                    nxt.append(vs[-1])
                vs = nxt
            return vs[0]

        def accumulate(par):
            def row(r, _):
                for c in range(NGRP):
                    sl = pl.ds(c * 16, 16)

                    def halves(j):
                        u = lax.bitcast_convert_type(
                            buf[par, j * CHUNK + r, sl], jnp.uint32)
                        return (lax.bitcast_convert_type(u << 16, jnp.float32),
                                lax.bitcast_convert_type(u, jnp.float32))

                    lo, hi = halves(0)
                    for j in range(1, MAXD):
                        l2, h2 = halves(j)
                        lo = lo + l2
                        hi = hi + h2
                    lo_u = (lax.bitcast_convert_type(lo, jnp.uint32) + RND) >> 16
                    hi_u = (lax.bitcast_convert_type(hi, jnp.uint32) + RND) & HMASK
                    obuf[par, r, sl] = lax.bitcast_convert_type(hi_u | lo_u, jnp.int32)
                return _
            lax.fori_loop(0, CHUNK, row, 0)

        def phase(i, s, par):
            nxt = s + 1

            @pl.when(nxt < NSUB)
            def _():
                fire(nxt, 1 - par, gsem[1 - par])

            with jax.named_scope("drain_g"):
                drain_gathers(par, gsem[par])

            @pl.when(i > 0)
            def _():
                drain_writeout(par, osem[par])

            with jax.named_scope("acc"):
                accumulate(par)
            pltpu.async_copy(obuf.at[par], out_hbm.at[pl.ds(base + s * CHUNK, CHUNK)],
                             osem[par])

        fire(0, 0, g0)

        def body(i, _):
            phase(i, 2 * i, 0)
            phase(i, 2 * i + 1, 1)
            return _

        lax.fori_loop(0, NSUB // 2, body, 0)
        drain_writeout(0, o0)
        drain_writeout(1, o1)

    return k(table, idx4)


def _pack_words(h):
    u = lax.bitcast_convert_type(h, jnp.uint32)
    lo = (u[:, :D // 2] + np.uint32(0x8000)) >> 16
    hi = (u[:, D // 2:] + np.uint32(0x8000)) & np.uint32(0xFFFF0000)
    return lax.bitcast_convert_type(hi | lo, jnp.int32)


def _unpack_words(w):
    u = lax.bitcast_convert_type(w, jnp.uint32)
    lo = lax.bitcast_convert_type(u << 16, jnp.float32)
    hi = lax.bitcast_convert_type(u & np.uint32(0xFFFF0000), jnp.float32)
    return jnp.concatenate([lo, hi], axis=1)


def _tc_pack_x(x_pad):
    def body(x_ref, o_ref):
        o_ref[...] = _pack_words(x_ref[...])

    return pl.pallas_call(
        body,
        grid=(NB,),
        in_specs=[pl.BlockSpec((RBLK, D), lambda i: (i, 0))],
        out_specs=pl.BlockSpec((RBLK, D // 2), lambda i: (i, 0)),
        out_shape=jax.ShapeDtypeStruct((NPAD, D // 2), jnp.int32),
    )(x_pad)


def _tc_layer1(agg1, x_pad, segT, W0, b0):
    def body(agg_ref, x_ref, segT_ref, w_ref, b_ref, h_ref, ge0_ref, ge1_ref):
        i = pl.program_id(0)
        agg = _unpack_words(agg_ref[...]).astype(jnp.bfloat16)
        h = jnp.dot(agg, w_ref[...].astype(jnp.bfloat16),
                    preferred_element_type=jnp.float32)
        h = jnp.maximum(h + b_ref[...], 0.0)
        h_ref[...] = _pack_words(h)
        s = segT_ref[...]
        p0 = jnp.dot(s, x_ref[...], preferred_element_type=jnp.float32)
        p1 = jnp.dot(s, h, preferred_element_type=jnp.float32)

        @pl.when(i == 0)
        def _():
            ge0_ref[...] = p0
            ge1_ref[...] = p1

        @pl.when(i > 0)
        def _():
            ge0_ref[...] += p0
            ge1_ref[...] += p1

    return pl.pallas_call(
        body,
        grid=(NB,),
        in_specs=[
            pl.BlockSpec((RBLK, D // 2), lambda i: (i, 0)),
            pl.BlockSpec((RBLK, D), lambda i: (i, 0)),
            pl.BlockSpec((G, RBLK), lambda i: (0, i)),
            pl.BlockSpec((D, HID), lambda i: (0, 0)),
            pl.BlockSpec((1, HID), lambda i: (0, 0)),
        ],
        out_specs=[
            pl.BlockSpec((RBLK, HID // 2), lambda i: (i, 0)),
            pl.BlockSpec((G, D), lambda i: (0, 0)),
            pl.BlockSpec((G, HID), lambda i: (0, 0)),
        ],
        out_shape=[
            jax.ShapeDtypeStruct((NPAD, HID // 2), jnp.int32),
            jax.ShapeDtypeStruct((G, D), jnp.float32),
            jax.ShapeDtypeStruct((G, HID), jnp.float32),
        ],
    )(agg1, x_pad, segT, W0, b0)


def _tc_layer2(agg2, segT, W1, b1, ge0, ge1, C0w, C1w, C2w, cb):
    def body(agg_ref, segT_ref, w_ref, b_ref, ge0_ref, ge1_ref,
             c0_ref, c1_ref, c2_ref, cb_ref, preds_ref, acc_ref):
        i = pl.program_id(0)
        agg = _unpack_words(agg_ref[...]).astype(jnp.bfloat16)
        h = jnp.dot(agg, w_ref[...].astype(jnp.bfloat16),
                    preferred_element_type=jnp.float32)
        h = jnp.maximum(h + b_ref[...], 0.0)
        p2 = jnp.dot(segT_ref[...], h, preferred_element_type=jnp.float32)

        @pl.when(i == 0)
        def _():
            acc_ref[...] = p2

        @pl.when(i > 0)
        def _():
            acc_ref[...] += p2

        @pl.when(i == NB - 1)
        def _():
            preds = jnp.dot(ge0_ref[...], c0_ref[...],
                            preferred_element_type=jnp.float32)
            preds += jnp.dot(ge1_ref[...], c1_ref[...],
                             preferred_element_type=jnp.float32)
            preds += jnp.dot(acc_ref[...], c2_ref[...],
                             preferred_element_type=jnp.float32)
            preds_ref[...] = preds + cb_ref[...]

    return pl.pallas_call(
        body,
        grid=(NB,),
        in_specs=[
            pl.BlockSpec((RBLK, HID // 2), lambda i: (i, 0)),
            pl.BlockSpec((G, RBLK), lambda i: (0, i)),
            pl.BlockSpec((HID, HID), lambda i: (0, 0)),
            pl.BlockSpec((1, HID), lambda i: (0, 0)),
            pl.BlockSpec((G, D), lambda i: (0, 0)),
            pl.BlockSpec((G, HID), lambda i: (0, 0)),
            pl.BlockSpec((D, OUT), lambda i: (0, 0)),
            pl.BlockSpec((HID, OUT), lambda i: (0, 0)),
            pl.BlockSpec((HID, OUT), lambda i: (0, 0)),
            pl.BlockSpec((1, OUT), lambda i: (0, 0)),
        ],
        out_specs=pl.BlockSpec((G, OUT), lambda i: (0, 0)),
        out_shape=jax.ShapeDtypeStruct((G, OUT), jnp.float32),
        scratch_shapes=[pltpu.VMEM((G, HID), jnp.float32)],
    )(agg2, segT, W1, b1, ge0, ge1, C0w, C1w, C2w, cb)


def kernel(x, neighbours, segment_ids, W0, b0, W1, b1,
           C0w, C0b, C1w, C1b, C2w, C2b):
    x_pad = jnp.zeros((NPAD, D), jnp.float32).at[:N].set(x)
    spread = (jnp.arange((NPAD - N) * MAXD, dtype=jnp.int32) * 37) % N
    nbr_pad = jnp.concatenate(
        [neighbours, spread.reshape(NPAD - N, MAXD)], axis=0)
    idxp = nbr_pad.reshape(NW, NSUB, CHUNK, MAXD).transpose(0, 1, 3, 2)
    idx3 = jnp.zeros((NW, NSUB, IDXROW), jnp.int32).at[:, :, :MAXD * CHUNK].set(
        idxp.reshape(NW, NSUB, MAXD * CHUNK))
    seg_pad = jnp.full((NPAD,), G, jnp.int32).at[:N].set(segment_ids)
    segT = (seg_pad[None, :] == jnp.arange(G, dtype=jnp.int32)[:, None]
            ).astype(jnp.float32)

    xw = _tc_pack_x(x_pad)
    agg1 = _sc_gather_sum(xw, idx3)
    H1w, ge0, ge1 = _tc_layer1(agg1, x_pad, segT, W0, b0.reshape(1, HID))
    agg2 = _sc_gather_sum(H1w, idx3)
    cb = (C0b + C1b + C2b).reshape(1, OUT)
    return _tc_layer2(agg2, segT, W1, b1.reshape(1, HID),
                      ge0, ge1, C0w, C1w, C2w, cb)


# speedup vs baseline: 4.1481x; 1.0137x over previous
import functools

import jax
import jax.numpy as jnp
import numpy as np
from jax import lax
from jax.experimental import pallas as pl
from jax.experimental.pallas import tpu as pltpu
from jax.experimental.pallas import tpu_sc as plsc

N = 10000
D = 256
HID = 256
OUT = 64
MAXD = 17
G = 8

NC = 2
NS = 16
NW = NC * NS
PW = 320
NPAD = NW * PW
CHUNK = 16
NSUB = PW // CHUNK
GLANES = 32
NGRP = D // GLANES
IDXROW = 512

RBLK = 1024
NB = NPAD // RBLK


def _sc_gather_sum(table, idx4):
    mesh = plsc.VectorSubcoreMesh(core_axis_name="c", subcore_axis_name="s")
    DW = D // 2

    @functools.partial(
        pl.kernel,
        out_type=jax.ShapeDtypeStruct((NPAD, DW), jnp.int32),
        mesh=mesh,
        scratch_types=[
            pltpu.VMEM((NSUB, IDXROW), jnp.int32),
            pltpu.VMEM((2, MAXD * CHUNK, DW), jnp.int32),
            pltpu.VMEM((2, CHUNK, DW), jnp.int32),
            pltpu.SemaphoreType.DMA,
            pltpu.SemaphoreType.DMA,
            pltpu.SemaphoreType.DMA,
            pltpu.SemaphoreType.DMA,
        ],
    )
    def k(table_hbm, idx_hbm, out_hbm, idx_v, buf, obuf, g0, g1, o0, o1):
        wid = lax.axis_index("s") * NC + lax.axis_index("c")
        base = wid * PW
        gsem = (g0, g1)
        osem = (o0, o1)
        pltpu.sync_copy(idx_hbm.at[wid], idx_v)

        SPLITS = ((0, 128), (128, 128), (256, MAXD * CHUNK - 256))

        def fire(s, par, sem):
            for lo, n in SPLITS:
                pltpu.async_copy(table_hbm.at[idx_v.at[s, pl.ds(lo, n)]],
                                 buf.at[par, pl.ds(lo, n)], sem)

        def drain_gathers(par, sem):
            for lo, n in SPLITS:
                pltpu.make_async_copy(table_hbm.at[pl.ds(0, n)],
                                      buf.at[par, pl.ds(lo, n)], sem).wait()

        def drain_writeout(par, sem):
            pltpu.make_async_copy(table_hbm.at[pl.ds(0, CHUNK)],
                                  obuf.at[par], sem).wait()

        HMASK = jnp.uint32(0xFFFF0000)
        RND = jnp.uint32(0x8000)

        def tree(vs):
            while len(vs) > 1:
                nxt = [vs[m] + vs[m + 1] for m in range(0, len(vs) - 1, 2)]
                if len(vs) % 2:
                    nxt.append(vs[-1])
                vs = nxt
            return vs[0]

        def accumulate(par):
            def row(r, _):
                for c in range(NGRP):
                    sl = pl.ds(c * 16, 16)

                    def halves(j):
                        u = lax.bitcast_convert_type(
                            buf[par, j * CHUNK + r, sl], jnp.uint32)
                        return (lax.bitcast_convert_type(u << 16, jnp.float32),
                                lax.bitcast_convert_type(u, jnp.float32))

                    lo, hi = halves(0)
                    for j in range(1, MAXD):
                        l2, h2 = halves(j)
                        lo = lo + l2
                        hi = hi + h2
                    lo_u = (lax.bitcast_convert_type(lo, jnp.uint32) + RND) >> 16
                    hi_u = (lax.bitcast_convert_type(hi, jnp.uint32) + RND) & HMASK
                    obuf[par, r, sl] = lax.bitcast_convert_type(hi_u | lo_u, jnp.int32)
                return _
            lax.fori_loop(0, CHUNK, row, 0)

        def phase(i, s, par):
            nxt = s + 1

            @pl.when(nxt < NSUB)
            def _():
                fire(nxt, 1 - par, gsem[1 - par])

            with jax.named_scope("drain_g"):
                drain_gathers(par, gsem[par])

            @pl.when(i > 0)
            def _():
                drain_writeout(par, osem[par])

            with jax.named_scope("acc"):
                accumulate(par)
            pltpu.async_copy(obuf.at[par], out_hbm.at[pl.ds(base + s * CHUNK, CHUNK)],
                             osem[par])

        fire(0, 0, g0)

        def body(i, _):
            phase(i, 2 * i, 0)
            phase(i, 2 * i + 1, 1)
            return _

        lax.fori_loop(0, NSUB // 2, body, 0)
        drain_writeout(0, o0)
        drain_writeout(1, o1)

    return k(table, idx4)


def _pack_words(h):
    u = lax.bitcast_convert_type(h, jnp.uint32)
    lo = (u[:, :D // 2] + np.uint32(0x8000)) >> 16
    hi = (u[:, D // 2:] + np.uint32(0x8000)) & np.uint32(0xFFFF0000)
    return lax.bitcast_convert_type(hi | lo, jnp.int32)


def _unpack_words(w):
    u = lax.bitcast_convert_type(w, jnp.uint32)
    lo = lax.bitcast_convert_type(u << 16, jnp.float32)
    hi = lax.bitcast_convert_type(u & np.uint32(0xFFFF0000), jnp.float32)
    return jnp.concatenate([lo, hi], axis=1)


RBLKX = 1000
NBX = N // RBLKX


def _tc_pack_x(x, segTx):
    def body(x_ref, segT_ref, o_ref, ge0_ref):
        i = pl.program_id(0)
        xb = x_ref[...]
        o_ref[...] = _pack_words(xb)
        p0 = lax.dot_general(segT_ref[...], xb, (((0,), (0,)), ((), ())),
                             preferred_element_type=jnp.float32)

        @pl.when(i == 0)
        def _():
            ge0_ref[...] = p0

        @pl.when(i > 0)
        def _():
            ge0_ref[...] += p0

    return pl.pallas_call(
        body,
        grid=(NBX,),
        in_specs=[
            pl.BlockSpec((RBLKX, D), lambda i: (i, 0)),
            pl.BlockSpec((RBLKX, G), lambda i: (i, 0)),
        ],
        out_specs=[
            pl.BlockSpec((RBLKX, D // 2), lambda i: (i, 0)),
            pl.BlockSpec((G, D), lambda i: (0, 0)),
        ],
        out_shape=[
            jax.ShapeDtypeStruct((N, D // 2), jnp.int32),
            jax.ShapeDtypeStruct((G, D), jnp.float32),
        ],
    )(x, segTx)


def _tc_layer1(agg1, segT, W0, b0):
    def body(agg_ref, segT_ref, w_ref, b_ref, h_ref, ge1_ref):
        i = pl.program_id(0)
        agg = _unpack_words(agg_ref[...]).astype(jnp.bfloat16)
        h = jnp.dot(agg, w_ref[...].astype(jnp.bfloat16),
                    preferred_element_type=jnp.float32)
        h = jnp.maximum(h + b_ref[...], 0.0)
        h_ref[...] = _pack_words(h)
        p1 = jnp.dot(segT_ref[...], h, preferred_element_type=jnp.float32)

        @pl.when(i == 0)
        def _():
            ge1_ref[...] = p1

        @pl.when(i > 0)
        def _():
            ge1_ref[...] += p1

    return pl.pallas_call(
        body,
        grid=(NB,),
        in_specs=[
            pl.BlockSpec((RBLK, D // 2), lambda i: (i, 0)),
            pl.BlockSpec((G, RBLK), lambda i: (0, i)),
            pl.BlockSpec((D, HID), lambda i: (0, 0)),
            pl.BlockSpec((1, HID), lambda i: (0, 0)),
        ],
        out_specs=[
            pl.BlockSpec((RBLK, HID // 2), lambda i: (i, 0)),
            pl.BlockSpec((G, HID), lambda i: (0, 0)),
        ],
        out_shape=[
            jax.ShapeDtypeStruct((NPAD, HID // 2), jnp.int32),
            jax.ShapeDtypeStruct((G, HID), jnp.float32),
        ],
    )(agg1, segT, W0, b0)


def _tc_layer2(agg2, segT, W1, b1, ge0, ge1, C0w, C1w, C2w, cb):
    def body(agg_ref, segT_ref, w_ref, b_ref, ge0_ref, ge1_ref,
             c0_ref, c1_ref, c2_ref, cb_ref, preds_ref, acc_ref):
        i = pl.program_id(0)
        agg = _unpack_words(agg_ref[...]).astype(jnp.bfloat16)
        h = jnp.dot(agg, w_ref[...].astype(jnp.bfloat16),
                    preferred_element_type=jnp.float32)
        h = jnp.maximum(h + b_ref[...], 0.0)
        p2 = jnp.dot(segT_ref[...], h, preferred_element_type=jnp.float32)

        @pl.when(i == 0)
        def _():
            acc_ref[...] = p2

        @pl.when(i > 0)
        def _():
            acc_ref[...] += p2

        @pl.when(i == NB - 1)
        def _():
            preds = jnp.dot(ge0_ref[...], c0_ref[...],
                            preferred_element_type=jnp.float32)
            preds += jnp.dot(ge1_ref[...], c1_ref[...],
                             preferred_element_type=jnp.float32)
            preds += jnp.dot(acc_ref[...], c2_ref[...],
                             preferred_element_type=jnp.float32)
            preds_ref[...] = preds + cb_ref[...]

    return pl.pallas_call(
        body,
        grid=(NB,),
        in_specs=[
            pl.BlockSpec((RBLK, HID // 2), lambda i: (i, 0)),
            pl.BlockSpec((G, RBLK), lambda i: (0, i)),
            pl.BlockSpec((HID, HID), lambda i: (0, 0)),
            pl.BlockSpec((1, HID), lambda i: (0, 0)),
            pl.BlockSpec((G, D), lambda i: (0, 0)),
            pl.BlockSpec((G, HID), lambda i: (0, 0)),
            pl.BlockSpec((D, OUT), lambda i: (0, 0)),
            pl.BlockSpec((HID, OUT), lambda i: (0, 0)),
            pl.BlockSpec((HID, OUT), lambda i: (0, 0)),
            pl.BlockSpec((1, OUT), lambda i: (0, 0)),
        ],
        out_specs=pl.BlockSpec((G, OUT), lambda i: (0, 0)),
        out_shape=jax.ShapeDtypeStruct((G, OUT), jnp.float32),
        scratch_shapes=[pltpu.VMEM((G, HID), jnp.float32)],
    )(agg2, segT, W1, b1, ge0, ge1, C0w, C1w, C2w, cb)


def kernel(x, neighbours, segment_ids, W0, b0, W1, b1,
           C0w, C0b, C1w, C1b, C2w, C2b):
    spread = (jnp.arange((NPAD - N) * MAXD, dtype=jnp.int32) * 37) % N
    nbr_pad = jnp.concatenate(
        [neighbours, spread.reshape(NPAD - N, MAXD)], axis=0)
    idxp = nbr_pad.reshape(NW, NSUB, CHUNK, MAXD).transpose(0, 1, 3, 2)
    idx3 = jnp.zeros((NW, NSUB, IDXROW), jnp.int32).at[:, :, :MAXD * CHUNK].set(
        idxp.reshape(NW, NSUB, MAXD * CHUNK))
    gids = jnp.arange(G, dtype=jnp.int32)[:, None]
    segTx = (segment_ids[:, None] == gids.T).astype(jnp.float32)
    seg_pad = jnp.full((NPAD,), G, jnp.int32).at[:N].set(segment_ids)
    segT = (seg_pad[None, :] == gids).astype(jnp.float32)

    xw, ge0 = _tc_pack_x(x, segTx)
    agg1 = _sc_gather_sum(xw, idx3)
    H1w, ge1 = _tc_layer1(agg1, segT, W0, b0.reshape(1, HID))
    agg2 = _sc_gather_sum(H1w, idx3)
    cb = (C0b + C1b + C2b).reshape(1, OUT)
    return _tc_layer2(agg2, segT, W1, b1.reshape(1, HID),
                      ge0, ge1, C0w, C1w, C2w, cb)
